# baseline (speedup 1.0000x reference)
import jax
import jax.numpy as jnp
from jax import lax
from jax.experimental import pallas as pl
from jax.experimental.pallas import tpu as pltpu

EMB = 32
QS = 64
SH = 64
L = 50
T = 32768
BLK = 2048
NB = T // BLK


def _dot(a, b, ca, cb):
    return lax.dot_general(a, b, (((ca,), (cb,)), ((), ())),
                           preferred_element_type=jnp.float32,
                           precision=lax.Precision.HIGHEST)


def _row_to_col(row):
    n = row.shape[1]
    sub = lax.broadcasted_iota(jnp.int32, (n, n), 0)
    lanes = lax.broadcasted_iota(jnp.int32, (n, n), 1)
    diag = (sub == lanes).astype(jnp.float32)
    return jnp.sum(row * diag, axis=1, keepdims=True)


def _copy_ques_kernel(question_ref, score_ref, hlast_ref, embT_hbm,
                      qhT_ref, hsT_ref,
                      WihT_f, WihT_b, bih_f, bih_b,
                      WhhT_f, WhhT_b, bhh_f, bhh_b,
                      gWih, gWhhT, gbih, gbhh,
                      qnT_out, hnT_out, q_out,
                      tiles_scr, gi_scr, wcat_scr, bhhcat_scr,
                      qcol_scr, hnewcol_scr, sem):
    i = pl.program_id(0)
    E = EMB

    def _cp(j):
        base = pl.multiple_of((question_ref[j] // 128) * 128, 128)
        return pltpu.make_async_copy(
            embT_hbm.at[:, pl.ds(base, 128)],
            tiles_scr.at[:, pl.ds(j * 128, 128)], sem)

    @pl.when(i == 0)
    def _():
        for j in range(L):
            _cp(j).start()

    @pl.when(i == 1)
    def _():
        zEE = jnp.zeros((E, E), jnp.float32)
        Wf = WhhT_f[...]
        Wb = WhhT_b[...]

        def two(a, b):
            return jnp.concatenate([a, b], axis=0)

        wcat_scr[...] = jnp.concatenate(
            [two(Wf[:, 0:E], zEE), two(zEE, Wb[:, 0:E]),
             two(Wf[:, E:2 * E], zEE), two(zEE, Wb[:, E:2 * E]),
             two(Wf[:, 2 * E:], zEE), two(zEE, Wb[:, 2 * E:])], axis=1)
        bf = bhh_f[...]
        bb = bhh_b[...]
        bhhcat_scr[...] = jnp.concatenate(
            [bf[:, 0:E], bb[:, 0:E], bf[:, E:2 * E], bb[:, E:2 * E],
             bf[:, 2 * E:], bb[:, 2 * E:]], axis=1)

    @pl.when(i == 2)
    def _():
        for j in range(L):
            _cp(j).wait()
        laneL = lax.broadcasted_iota(jnp.int32, (1, L), 1)
        r_f = jnp.zeros((1, L), jnp.int32)
        r_b = jnp.zeros((1, L), jnp.int32)
        for j in range(L):
            r_f = jnp.where(laneL == j, question_ref[j] % 128, r_f)
            r_b = jnp.where(laneL == j, question_ref[L - 1 - j] % 128, r_b)
        k_iota = lax.broadcasted_iota(jnp.int32, (L * 128, L), 0)
        j_iota = lax.broadcasted_iota(jnp.int32, (L * 128, L), 1)
        tile_of_k = k_iota // 128
        lane_of_k = k_iota % 128
        sel_f = ((tile_of_k == j_iota)
                 & (lane_of_k == r_f)).astype(jnp.float32)
        sel_b = ((tile_of_k == (L - 1 - j_iota))
                 & (lane_of_k == r_b)).astype(jnp.float32)
        tiles = tiles_scr[...]
        x_cols = _dot(tiles, sel_f, 1, 0)
        x_cols_rev = _dot(tiles, sel_b, 1, 0)

        gif = _dot(x_cols, WihT_f[...], 0, 0) + bih_f[...]
        gib = _dot(x_cols_rev, WihT_b[...], 0, 0) + bih_b[...]
        gi_scr[...] = jnp.concatenate(
            [gif[:, 0:E], gib[:, 0:E], gif[:, E:2 * E], gib[:, E:2 * E],
             gif[:, 2 * E:], gib[:, 2 * E:]], axis=1)

        def step(t, carry):
            h, m = carry
            gi = gi_scr[pl.ds(t, 1), :]
            gh = _dot(h, wcat_scr[...], 1, 0) + bhhcat_scr[...]
            rz = jax.nn.sigmoid(gi[:, :4 * E] + gh[:, :4 * E])
            r, z = rz[:, :2 * E], rz[:, 2 * E:]
            n = jnp.tanh(gi[:, 4 * E:] + r * gh[:, 4 * E:])
            h = (1.0 - z) * n + z * h
            return (h, jnp.maximum(m, h))

        zeros = jnp.zeros((1, 2 * E), jnp.float32)
        ninf = jnp.full((1, 2 * E), -jnp.inf, jnp.float32)
        _, q = lax.fori_loop(0, L, step, (zeros, ninf))
        q_out[...] = q
        qcol_scr[...] = _row_to_col(q)

        s = score_ref[0]
        pos = (s >= 0.5).astype(jnp.float32)
        x_in = jnp.concatenate([q * pos, q * (1.0 - pos)], axis=1)
        gi = _dot(x_in, gWih[...], 1, 1) + gbih[...]
        gh = _dot(hlast_ref[...], gWhhT[...], 1, 0) + gbhh[...]
        H = SH
        r = jax.nn.sigmoid(gi[:, :H] + gh[:, :H])
        z = jax.nn.sigmoid(gi[:, H:2 * H] + gh[:, H:2 * H])
        n = jnp.tanh(gi[:, 2 * H:] + r * gh[:, 2 * H:])
        hnewcol_scr[...] = _row_to_col((1.0 - z) * n + z * hlast_ref[...])

    @pl.when(i < NB)
    def _():
        qnT_out[...] = qhT_ref[...]
        hnT_out[...] = hsT_ref[...]

    @pl.when(i == NB)
    def _():
        qnT_out[:, pl.ds(0, 1)] = qcol_scr[...]
        hnT_out[:, pl.ds(0, 1)] = hnewcol_scr[...]


def _attend_kernel(qhT_ref, hsT_any, q_ref, sW_ref, sb_ref,
                   pred_out, run_max, gidx, tile_scr, sem):
    i = pl.program_id(0)

    @pl.when(i == 0)
    def _():
        run_max[0] = -jnp.inf
        gidx[0] = 0

    alpha = _dot(q_ref[...], qhT_ref[...], 1, 0)
    m = jnp.max(alpha)

    @pl.when(m > run_max[0])
    def _():
        run_max[0] = m
        lanes = lax.broadcasted_iota(jnp.int32, (1, BLK), 1)
        a = jnp.min(jnp.where(alpha >= m, lanes, BLK))
        gidx[0] = i * BLK + a

    @pl.when(i == NB - 1)
    def _():
        g = gidx[0]
        base = pl.multiple_of((g // 128) * 128, 128)
        cp = pltpu.make_async_copy(hsT_any.at[:, pl.ds(base, 128)],
                                   tile_scr, sem)
        cp.start()
        cp.wait()
        lane128 = lax.broadcasted_iota(jnp.int32, (1, 128), 1)
        onehot = (lane128 == g % 128).astype(jnp.float32)
        attn_col = jnp.sum(tile_scr[...] * onehot, axis=1, keepdims=True)
        t1 = jnp.sum(q_ref[...] * sW_ref[:, :QS])
        t2 = jnp.sum(attn_col * _row_to_col(sW_ref[:, QS:]))
        pred_out[...] = jnp.zeros((1, 1), jnp.float32) + t1 + t2 + sb_ref[0]


def kernel(question, score, questions_hist, hs_hist, emb,
           qWih_f, qWhh_f, qbih_f, qbhh_f,
           qWih_b, qWhh_b, qbih_b, qbhh_b,
           sW, sb, gWih, gWhh, gbih, gbhh):
    question = question.astype(jnp.int32)
    f32 = jnp.float32
    E = EMB

    qhT = questions_hist.T
    hsT = jnp.transpose(hs_hist, (1, 2, 0)).reshape(SH, T)
    embT = emb.T
    hlast = hs_hist[T - 1].reshape(1, SH)

    qnT, hnT, q = pl.pallas_call(
        _copy_ques_kernel,
        grid=(NB + 1,),
        in_specs=[
            pl.BlockSpec(memory_space=pltpu.MemorySpace.SMEM),
            pl.BlockSpec(memory_space=pltpu.MemorySpace.SMEM),
            pl.BlockSpec(memory_space=pltpu.MemorySpace.VMEM),
            pl.BlockSpec(memory_space=pltpu.MemorySpace.HBM),
            pl.BlockSpec((QS, BLK), lambda i: (0, jnp.minimum(i, NB - 1))),
            pl.BlockSpec((SH, BLK), lambda i: (0, jnp.minimum(i, NB - 1))),
        ] + [pl.BlockSpec(memory_space=pltpu.MemorySpace.VMEM)] * 12,
        out_specs=[
            pl.BlockSpec((QS, BLK), lambda i: (0, i)),
            pl.BlockSpec((SH, BLK), lambda i: (0, i)),
            pl.BlockSpec((1, QS), lambda i: (0, 0)),
        ],
        out_shape=[
            jax.ShapeDtypeStruct((QS, T + 1), f32),
            jax.ShapeDtypeStruct((SH, T + 1), f32),
            jax.ShapeDtypeStruct((1, QS), f32),
        ],
        scratch_shapes=[pltpu.VMEM((EMB, L * 128), f32),
                        pltpu.VMEM((L, 6 * E), f32),
                        pltpu.VMEM((2 * E, 6 * E), f32),
                        pltpu.VMEM((1, 6 * E), f32),
                        pltpu.VMEM((QS, 1), f32),
                        pltpu.VMEM((SH, 1), f32),
                        pltpu.SemaphoreType.DMA],
    )(question, score.astype(f32), hlast, embT, qhT, hsT,
      qWih_f.T, qWih_b.T, qbih_f.reshape(1, -1), qbih_b.reshape(1, -1),
      qWhh_f.T, qWhh_b.T, qbhh_f.reshape(1, -1), qbhh_b.reshape(1, -1),
      gWih, gWhh.T, gbih.reshape(1, -1), gbhh.reshape(1, -1))

    pred = pl.pallas_call(
        _attend_kernel,
        grid=(NB,),
        in_specs=[
            pl.BlockSpec((QS, BLK), lambda i: (0, i)),
            pl.BlockSpec(memory_space=pltpu.MemorySpace.HBM),
            pl.BlockSpec((1, QS), lambda i: (0, 0)),
            pl.BlockSpec((1, QS + SH), lambda i: (0, 0)),
            pl.BlockSpec(memory_space=pltpu.MemorySpace.SMEM),
        ],
        out_specs=pl.BlockSpec((1, 1), lambda i: (0, 0)),
        out_shape=jax.ShapeDtypeStruct((1, 1), f32),
        scratch_shapes=[pltpu.SMEM((1,), f32), pltpu.SMEM((1,), jnp.int32),
                        pltpu.VMEM((SH, 128), f32),
                        pltpu.SemaphoreType.DMA],
    )(qhT, hsT, q, sW, sb.astype(f32))

    qn = qnT.T
    hn = jnp.transpose(hnT.reshape(1, SH, T + 1), (2, 0, 1))
    return pred, qn, hn


# speedup vs baseline: 1.1093x; 1.1093x over previous
import jax
import jax.numpy as jnp
from jax import lax
from jax.experimental import pallas as pl
from jax.experimental.pallas import tpu as pltpu

EMB = 32
QS = 64
SH = 64
L = 50
T = 32768
BLK = 2048
NB = T // BLK


def _dot(a, b, ca, cb):
    return lax.dot_general(a, b, (((ca,), (cb,)), ((), ())),
                           preferred_element_type=jnp.float32)


def _row_to_col(row):
    n = row.shape[1]
    sub = lax.broadcasted_iota(jnp.int32, (n, n), 0)
    lanes = lax.broadcasted_iota(jnp.int32, (n, n), 1)
    diag = (sub == lanes).astype(jnp.float32)
    return jnp.sum(row * diag, axis=1, keepdims=True)


def _copy_ques_kernel(question_ref, score_ref, hlast_ref, embT_hbm,
                      qhT_ref, hsT_ref,
                      WihT_f, WihT_b, bih_f, bih_b,
                      WhhT_f, WhhT_b, bhh_f, bhh_b,
                      gWih, gWhhT, gbih, gbhh,
                      qnT_out, hnT_out, q_out,
                      tiles_scr, gi_scr, wcat_scr, bhhcat_scr,
                      qcol_scr, hnewcol_scr, sem):
    i = pl.program_id(0)
    E = EMB

    def _cp(j):
        base = pl.multiple_of((question_ref[j] // 128) * 128, 128)
        return pltpu.make_async_copy(
            embT_hbm.at[:, pl.ds(base, 128)],
            tiles_scr.at[:, pl.ds(j * 128, 128)], sem)

    @pl.when(i == 0)
    def _():
        for j in range(L):
            _cp(j).start()

    @pl.when(i == 1)
    def _():
        zEE = jnp.zeros((E, E), jnp.float32)
        Wf = WhhT_f[...]
        Wb = WhhT_b[...]

        def two(a, b):
            return jnp.concatenate([a, b], axis=0)

        wcat_scr[...] = jnp.concatenate(
            [two(Wf[:, 0:E], zEE), two(zEE, Wb[:, 0:E]),
             two(Wf[:, E:2 * E], zEE), two(zEE, Wb[:, E:2 * E]),
             two(Wf[:, 2 * E:], zEE), two(zEE, Wb[:, 2 * E:])], axis=1)
        bf = bhh_f[...]
        bb = bhh_b[...]
        bhhcat_scr[...] = jnp.concatenate(
            [bf[:, 0:E], bb[:, 0:E], bf[:, E:2 * E], bb[:, E:2 * E],
             bf[:, 2 * E:], bb[:, 2 * E:]], axis=1)

    @pl.when(i < NB)
    def _():
        qnT_out[...] = qhT_ref[...]
        hnT_out[...] = hsT_ref[...]

    @pl.when(i == NB)
    def _():
        for j in range(L):
            _cp(j).wait()
        laneL = lax.broadcasted_iota(jnp.int32, (1, L), 1)
        r_f = jnp.zeros((1, L), jnp.int32)
        r_b = jnp.zeros((1, L), jnp.int32)
        for j in range(L):
            r_f = jnp.where(laneL == j, question_ref[j] % 128, r_f)
            r_b = jnp.where(laneL == j, question_ref[L - 1 - j] % 128, r_b)
        k_iota = lax.broadcasted_iota(jnp.int32, (L * 128, L), 0)
        j_iota = lax.broadcasted_iota(jnp.int32, (L * 128, L), 1)
        tile_of_k = k_iota // 128
        lane_of_k = k_iota % 128
        sel_f = ((tile_of_k == j_iota)
                 & (lane_of_k == r_f)).astype(jnp.float32)
        sel_b = ((tile_of_k == (L - 1 - j_iota))
                 & (lane_of_k == r_b)).astype(jnp.float32)
        tiles = tiles_scr[...]
        x_cols = _dot(tiles, sel_f, 1, 0)
        x_cols_rev = _dot(tiles, sel_b, 1, 0)

        gif = _dot(x_cols, WihT_f[...], 0, 0) + bih_f[...]
        gib = _dot(x_cols_rev, WihT_b[...], 0, 0) + bih_b[...]
        gi_scr[...] = jnp.concatenate(
            [gif[:, 0:E], gib[:, 0:E], gif[:, E:2 * E], gib[:, E:2 * E],
             gif[:, 2 * E:], gib[:, 2 * E:]], axis=1)

        def step(t, carry):
            h, m = carry
            gi = gi_scr[pl.ds(t, 1), :]
            gh = _dot(h, wcat_scr[...], 1, 0) + bhhcat_scr[...]
            rz = jax.nn.sigmoid(gi[:, :4 * E] + gh[:, :4 * E])
            r, z = rz[:, :2 * E], rz[:, 2 * E:]
            n = jnp.tanh(gi[:, 4 * E:] + r * gh[:, 4 * E:])
            h = (1.0 - z) * n + z * h
            return (h, jnp.maximum(m, h))

        zeros = jnp.zeros((1, 2 * E), jnp.float32)
        ninf = jnp.full((1, 2 * E), -jnp.inf, jnp.float32)
        _, q = lax.fori_loop(0, L, step, (zeros, ninf))
        q_out[...] = q
        qnT_out[:, pl.ds(0, 1)] = _row_to_col(q)

        s = score_ref[0]
        pos = (s >= 0.5).astype(jnp.float32)
        x_in = jnp.concatenate([q * pos, q * (1.0 - pos)], axis=1)
        gi = _dot(x_in, gWih[...], 1, 1) + gbih[...]
        gh = _dot(hlast_ref[...], gWhhT[...], 1, 0) + gbhh[...]
        H = SH
        r = jax.nn.sigmoid(gi[:, :H] + gh[:, :H])
        z = jax.nn.sigmoid(gi[:, H:2 * H] + gh[:, H:2 * H])
        n = jnp.tanh(gi[:, 2 * H:] + r * gh[:, 2 * H:])
        hnT_out[:, pl.ds(0, 1)] = _row_to_col(
            (1.0 - z) * n + z * hlast_ref[...])


def _attend_kernel(qhT_ref, hsT_any, q_ref, sW_ref, sb_ref,
                   pred_out, run_max, gidx, tile_scr, sem):
    i = pl.program_id(0)

    @pl.when(i == 0)
    def _():
        run_max[0] = -jnp.inf
        gidx[0] = 0

    alpha = _dot(q_ref[...], qhT_ref[...], 1, 0)
    m = jnp.max(alpha)

    @pl.when(m > run_max[0])
    def _():
        run_max[0] = m
        lanes = lax.broadcasted_iota(jnp.int32, (1, BLK), 1)
        a = jnp.min(jnp.where(alpha >= m, lanes, BLK))
        gidx[0] = i * BLK + a

    @pl.when(i == NB - 1)
    def _():
        g = gidx[0]
        base = pl.multiple_of((g // 128) * 128, 128)
        cp = pltpu.make_async_copy(hsT_any.at[:, pl.ds(base, 128)],
                                   tile_scr, sem)
        cp.start()
        cp.wait()
        lane128 = lax.broadcasted_iota(jnp.int32, (1, 128), 1)
        onehot = (lane128 == g % 128).astype(jnp.float32)
        attn_col = jnp.sum(tile_scr[...] * onehot, axis=1, keepdims=True)
        t1 = jnp.sum(q_ref[...] * sW_ref[:, :QS])
        t2 = jnp.sum(attn_col * _row_to_col(sW_ref[:, QS:]))
        pred_out[...] = jnp.zeros((1, 1), jnp.float32) + t1 + t2 + sb_ref[0]


def kernel(question, score, questions_hist, hs_hist, emb,
           qWih_f, qWhh_f, qbih_f, qbhh_f,
           qWih_b, qWhh_b, qbih_b, qbhh_b,
           sW, sb, gWih, gWhh, gbih, gbhh):
    question = question.astype(jnp.int32)
    f32 = jnp.float32
    E = EMB

    qhT = questions_hist.T
    hsT = jnp.transpose(hs_hist, (1, 2, 0)).reshape(SH, T)
    embT = emb.T
    hlast = hs_hist[T - 1].reshape(1, SH)

    qnT, hnT, q = pl.pallas_call(
        _copy_ques_kernel,
        grid=(NB + 1,),
        in_specs=[
            pl.BlockSpec(memory_space=pltpu.MemorySpace.SMEM),
            pl.BlockSpec(memory_space=pltpu.MemorySpace.SMEM),
            pl.BlockSpec(memory_space=pltpu.MemorySpace.VMEM),
            pl.BlockSpec(memory_space=pltpu.MemorySpace.HBM),
            pl.BlockSpec((QS, BLK), lambda i: (0, jnp.minimum(i, NB - 1))),
            pl.BlockSpec((SH, BLK), lambda i: (0, jnp.minimum(i, NB - 1))),
        ] + [pl.BlockSpec(memory_space=pltpu.MemorySpace.VMEM)] * 12,
        out_specs=[
            pl.BlockSpec((QS, BLK), lambda i: (0, i)),
            pl.BlockSpec((SH, BLK), lambda i: (0, i)),
            pl.BlockSpec((1, QS), lambda i: (0, 0)),
        ],
        out_shape=[
            jax.ShapeDtypeStruct((QS, T + 1), f32),
            jax.ShapeDtypeStruct((SH, T + 1), f32),
            jax.ShapeDtypeStruct((1, QS), f32),
        ],
        scratch_shapes=[pltpu.VMEM((EMB, L * 128), f32),
                        pltpu.VMEM((L, 6 * E), f32),
                        pltpu.VMEM((2 * E, 6 * E), f32),
                        pltpu.VMEM((1, 6 * E), f32),
                        pltpu.VMEM((QS, 1), f32),
                        pltpu.VMEM((SH, 1), f32),
                        pltpu.SemaphoreType.DMA],
    )(question, score.astype(f32), hlast, embT, qhT, hsT,
      qWih_f.T, qWih_b.T, qbih_f.reshape(1, -1), qbih_b.reshape(1, -1),
      qWhh_f.T, qWhh_b.T, qbhh_f.reshape(1, -1), qbhh_b.reshape(1, -1),
      gWih, gWhh.T, gbih.reshape(1, -1), gbhh.reshape(1, -1))

    pred = pl.pallas_call(
        _attend_kernel,
        grid=(NB,),
        in_specs=[
            pl.BlockSpec((QS, BLK), lambda i: (0, i)),
            pl.BlockSpec(memory_space=pltpu.MemorySpace.HBM),
            pl.BlockSpec((1, QS), lambda i: (0, 0)),
            pl.BlockSpec((1, QS + SH), lambda i: (0, 0)),
            pl.BlockSpec(memory_space=pltpu.MemorySpace.SMEM),
        ],
        out_specs=pl.BlockSpec((1, 1), lambda i: (0, 0)),
        out_shape=jax.ShapeDtypeStruct((1, 1), f32),
        scratch_shapes=[pltpu.SMEM((1,), f32), pltpu.SMEM((1,), jnp.int32),
                        pltpu.VMEM((SH, 128), f32),
                        pltpu.SemaphoreType.DMA],
    )(qhT, hsT, q, sW, sb.astype(f32))

    qn = qnT.T
    hn = jnp.transpose(hnT.reshape(1, SH, T + 1), (2, 0, 1))
    return pred, qn, hn
